# Initial kernel scaffold; baseline (speedup 1.0000x reference)
import functools

import jax
import jax.numpy as jnp
from jax import lax
from jax.experimental import pallas as pl
from jax.experimental.pallas import tpu as pltpu
from jax.experimental.pallas import tpu_sc as plsc

_EPS = 1e-5


def _dense_call(h, lsW, fsW, linsW, filmsW, filmsb, *, bn):
    n, d = h.shape
    hdim = lsW.shape[1]
    r_cnt = linsW.shape[0]

    def body(h_ref, ls_ref, fs_ref, lin_ref, film_ref, fb_ref,
             skip_ref, xt_ref, f_ref):
        hb = h_ref[...]
        ls = jnp.dot(hb, ls_ref[...], preferred_element_type=jnp.float32)
        fs = jnp.dot(hb, fs_ref[...], preferred_element_type=jnp.float32)
        skip_ref[...] = jnp.maximum(fs[:, hdim:] * ls + fs[:, :hdim], 0.0)
        for r in range(r_cnt):
            xt_ref[r] = jnp.dot(hb, lin_ref[r],
                                preferred_element_type=jnp.float32)
            f_ref[r] = jnp.dot(hb, film_ref[r],
                               preferred_element_type=jnp.float32) + fb_ref[r]

    return pl.pallas_call(
        body,
        grid=(n // bn,),
        in_specs=[
            pl.BlockSpec((bn, d), lambda i: (i, 0)),
            pl.BlockSpec((d, hdim), lambda i: (0, 0)),
            pl.BlockSpec((d, 2 * hdim), lambda i: (0, 0)),
            pl.BlockSpec((r_cnt, d, hdim), lambda i: (0, 0, 0)),
            pl.BlockSpec((r_cnt, d, 2 * hdim), lambda i: (0, 0, 0)),
            pl.BlockSpec((r_cnt, 2 * hdim), lambda i: (0, 0)),
        ],
        out_specs=[
            pl.BlockSpec((bn, hdim), lambda i: (i, 0)),
            pl.BlockSpec((r_cnt, bn, hdim), lambda i: (0, i, 0)),
            pl.BlockSpec((r_cnt, bn, 2 * hdim), lambda i: (0, i, 0)),
        ],
        out_shape=[
            jax.ShapeDtypeStruct((n, hdim), jnp.float32),
            jax.ShapeDtypeStruct((r_cnt, n, hdim), jnp.float32),
            jax.ShapeDtypeStruct((r_cnt, n, 2 * hdim), jnp.float32),
        ],
    )(h, lsW, fsW, linsW, filmsW, filmsb)


def _edge_call(src, dst, et, xt_flat, f_flat, *, n_nodes, r_cnt, hdim, k_chunk):
    e_cnt = src.shape[0]
    info = plsc.get_sparse_core_info()
    ns = info.num_subcores
    lanes = info.num_lanes

    np_rows = ((n_nodes + 1 + ns - 1) // ns) * ns
    rows_per_tile = np_rows // ns
    ep = e_cnt // ns
    n_chunks = ep // k_chunk
    assert ep % k_chunk == 0 and k_chunk % lanes == 0

    z_big = jnp.zeros((np_rows, hdim), jnp.float32)
    z_small = jnp.zeros((np_rows, lanes), jnp.float32)

    mesh = plsc.VectorSubcoreMesh(core_axis_name="c", subcore_axis_name="s")

    @functools.partial(
        pl.kernel,
        out_type=[
            jax.ShapeDtypeStruct((r_cnt, np_rows, hdim), jnp.float32),
            jax.ShapeDtypeStruct((r_cnt, np_rows, lanes), jnp.float32),
        ],
        mesh=mesh,
        scratch_types=[
            pltpu.VMEM_SHARED((np_rows, hdim), jnp.float32),
            pltpu.VMEM_SHARED((np_rows, lanes), jnp.float32),
            pltpu.VMEM((k_chunk,), jnp.int32),
            pltpu.VMEM((k_chunk,), jnp.int32),
            pltpu.VMEM((k_chunk,), jnp.int32),
            pltpu.VMEM((k_chunk,), jnp.int32),
            pltpu.VMEM((k_chunk,), jnp.int32),
            pltpu.VMEM((k_chunk,), jnp.int32),
            pltpu.VMEM((k_chunk, hdim), jnp.float32),
            pltpu.VMEM((k_chunk, 2 * hdim), jnp.float32),
            pltpu.VMEM((k_chunk, lanes), jnp.float32),
        ],
    )
    def edge_kernel(src_h, dst_h, et_h, xt_h, f_h, zb_h, zs_h,
                    ssum_h, cnt_h,
                    acc, cacc, sv, dv, ev, ixv, ifv, iov, xbuf, fbuf, ones_b):
        c = lax.axis_index("c")
        s = lax.axis_index("s")
        row0 = s * rows_per_tile

        pltpu.sync_copy(zb_h.at[pl.ds(row0, rows_per_tile)],
                        acc.at[pl.ds(row0, rows_per_tile)])
        pltpu.sync_copy(zs_h.at[pl.ds(row0, rows_per_tile)],
                        cacc.at[pl.ds(row0, rows_per_tile)])

        def fill_ones(i, carry):
            ones_b[i] = jnp.full((lanes,), 1.0, jnp.float32)
            return carry
        lax.fori_loop(0, k_chunk, fill_ones, 0)

        plsc.subcore_barrier()

        base = s * ep
        c_n = c * n_nodes

        def chunk(kk, carry):
            off = base + kk * k_chunk
            pltpu.sync_copy(src_h.at[pl.ds(off, k_chunk)], sv)
            pltpu.sync_copy(dst_h.at[pl.ds(off, k_chunk)], dv)
            pltpu.sync_copy(et_h.at[pl.ds(off, k_chunk)], ev)

            def idx_grp(g, carry2):
                sl = pl.ds(g * lanes, lanes)
                s16 = sv[sl]
                d16 = dv[sl]
                e16 = ev[sl]
                ixv[sl] = s16 + c_n
                ifv[sl] = d16 + c_n
                iov[sl] = jnp.where(e16 == c, d16, n_nodes)
                return carry2
            lax.fori_loop(0, k_chunk // lanes, idx_grp, 0)

            pltpu.sync_copy(xt_h.at[ixv], xbuf)
            pltpu.sync_copy(f_h.at[ifv], fbuf)

            def msg(e, carry2):
                for j in range(hdim // lanes):
                    sl = pl.ds(j * lanes, lanes)
                    x16 = xbuf[e, sl]
                    b16 = fbuf[e, sl]
                    g16 = fbuf[e, pl.ds(hdim + j * lanes, lanes)]
                    xbuf[e, sl] = jnp.maximum(g16 * x16 + b16, 0.0)
                return carry2
            lax.fori_loop(0, k_chunk, msg, 0)

            pltpu.sync_copy(xbuf, acc.at[iov], add=True)
            pltpu.sync_copy(ones_b, cacc.at[iov], add=True)
            return carry
        lax.fori_loop(0, n_chunks, chunk, 0)

        plsc.subcore_barrier()

        pltpu.sync_copy(acc.at[pl.ds(row0, rows_per_tile)],
                        ssum_h.at[c, pl.ds(row0, rows_per_tile)])
        pltpu.sync_copy(cacc.at[pl.ds(row0, rows_per_tile)],
                        cnt_h.at[c, pl.ds(row0, rows_per_tile)])

    return edge_kernel(src, dst, et, xt_flat, f_flat, z_big, z_small)


def _combine_call(skip, ssum, cnt, gamma, beta, rm, rv, *, bn):
    n, hdim = skip.shape
    r_cnt, np_rows, lanes = cnt.shape

    def body(skip_ref, ss_ref, ct_ref, g_ref, b_ref, rm_ref, rv_ref, out_ref):
        out = skip_ref[...]
        for r in range(r_cnt):
            c = ct_ref[r][:, 0:1]
            out = out + ss_ref[r] * (1.0 / jnp.maximum(c, 1.0))
        scale = g_ref[...] * lax.rsqrt(rv_ref[...] + _EPS)
        out_ref[...] = (out - rm_ref[...]) * scale + b_ref[...]

    return pl.pallas_call(
        body,
        grid=(n // bn,),
        in_specs=[
            pl.BlockSpec((bn, hdim), lambda i: (i, 0)),
            pl.BlockSpec((r_cnt, bn, hdim), lambda i: (0, i, 0)),
            pl.BlockSpec((r_cnt, bn, lanes), lambda i: (0, i, 0)),
            pl.BlockSpec((1, hdim), lambda i: (0, 0)),
            pl.BlockSpec((1, hdim), lambda i: (0, 0)),
            pl.BlockSpec((1, hdim), lambda i: (0, 0)),
            pl.BlockSpec((1, hdim), lambda i: (0, 0)),
        ],
        out_specs=pl.BlockSpec((bn, hdim), lambda i: (i, 0)),
        out_shape=jax.ShapeDtypeStruct((n, hdim), jnp.float32),
    )(skip, ssum, cnt, gamma, beta, rm, rv)


def _mlp_call(h, w1, b1, w2, b2, *, bn):
    n, hdim = h.shape
    mid = w1.shape[1]

    def body(h_ref, w1_ref, b1_ref, w2_ref, b2_ref, out_ref):
        y = jnp.dot(h_ref[...], w1_ref[...],
                    preferred_element_type=jnp.float32) + b1_ref[...]
        y = jnp.where(y > 0, y, 0.2 * y)
        out_ref[...] = jnp.dot(y, w2_ref[...],
                               preferred_element_type=jnp.float32) + b2_ref[...]

    return pl.pallas_call(
        body,
        grid=(n // bn,),
        in_specs=[
            pl.BlockSpec((bn, hdim), lambda i: (i, 0)),
            pl.BlockSpec((hdim, mid), lambda i: (0, 0)),
            pl.BlockSpec((1, mid), lambda i: (0, 0)),
            pl.BlockSpec((mid, hdim), lambda i: (0, 0)),
            pl.BlockSpec((1, hdim), lambda i: (0, 0)),
        ],
        out_specs=pl.BlockSpec((bn, hdim), lambda i: (i, 0)),
        out_shape=jax.ShapeDtypeStruct((n, hdim), jnp.float32),
    )(h, w1, b1, w2, b2)


def kernel(x, edge_index, edge_type, lins_W, films_W, films_b, lin_skip_W,
           film_skip_W, bn_gamma, bn_beta, bn_rm, bn_rv, lin1_W, lin1_b,
           lin2_W, lin2_b):
    n, d = x.shape
    l_cnt, r_cnt, _, hdim = lins_W.shape
    src = edge_index[0]
    dst = edge_index[1]
    bn = 2000

    h = x
    for l in range(l_cnt):
        skip, xt, f = _dense_call(
            h, lin_skip_W[l], film_skip_W[l], lins_W[l], films_W[l],
            films_b[l], bn=bn)
        ssum, cnt = _edge_call(
            src, dst, edge_type,
            xt.reshape(r_cnt * n, hdim), f.reshape(r_cnt * n, 2 * hdim),
            n_nodes=n, r_cnt=r_cnt, hdim=hdim, k_chunk=80)
        h = _combine_call(
            skip, ssum, cnt,
            bn_gamma[l].reshape(1, hdim), bn_beta[l].reshape(1, hdim),
            bn_rm[l].reshape(1, hdim), bn_rv[l].reshape(1, hdim), bn=bn)
    return _mlp_call(h, lin1_W, lin1_b.reshape(1, -1),
                     lin2_W, lin2_b.reshape(1, -1), bn=bn)


# speedup vs baseline: 2.2963x; 2.2963x over previous
import functools

import jax
import jax.numpy as jnp
from jax import lax
from jax.experimental import pallas as pl
from jax.experimental.pallas import tpu as pltpu
from jax.experimental.pallas import tpu_sc as plsc

_EPS = 1e-5


def _dense_call(h, lsW, fsW, linsW, filmsW, filmsb, *, bn):
    n, d = h.shape
    hdim = lsW.shape[1]
    r_cnt = linsW.shape[0]

    def body(h_ref, ls_ref, fs_ref, lin_ref, film_ref, fb_ref,
             skip_ref, xt_ref, f_ref):
        hb = h_ref[...]
        ls = jnp.dot(hb, ls_ref[...], preferred_element_type=jnp.float32)
        fs = jnp.dot(hb, fs_ref[...], preferred_element_type=jnp.float32)
        skip_ref[...] = jnp.maximum(fs[:, hdim:] * ls + fs[:, :hdim], 0.0)
        for r in range(r_cnt):
            xt_ref[r] = jnp.dot(hb, lin_ref[r],
                                preferred_element_type=jnp.float32)
            f_ref[r] = jnp.dot(hb, film_ref[r],
                               preferred_element_type=jnp.float32) + fb_ref[r]

    return pl.pallas_call(
        body,
        grid=(n // bn,),
        in_specs=[
            pl.BlockSpec((bn, d), lambda i: (i, 0)),
            pl.BlockSpec((d, hdim), lambda i: (0, 0)),
            pl.BlockSpec((d, 2 * hdim), lambda i: (0, 0)),
            pl.BlockSpec((r_cnt, d, hdim), lambda i: (0, 0, 0)),
            pl.BlockSpec((r_cnt, d, 2 * hdim), lambda i: (0, 0, 0)),
            pl.BlockSpec((r_cnt, 2 * hdim), lambda i: (0, 0)),
        ],
        out_specs=[
            pl.BlockSpec((bn, hdim), lambda i: (i, 0)),
            pl.BlockSpec((r_cnt, bn, hdim), lambda i: (0, i, 0)),
            pl.BlockSpec((r_cnt, bn, 2 * hdim), lambda i: (0, i, 0)),
        ],
        out_shape=[
            jax.ShapeDtypeStruct((n, hdim), jnp.float32),
            jax.ShapeDtypeStruct((r_cnt, n, hdim), jnp.float32),
            jax.ShapeDtypeStruct((r_cnt, n, 2 * hdim), jnp.float32),
        ],
    )(h, lsW, fsW, linsW, filmsW, filmsb)


_NS = 16
_LANES = 16


def _pad_rows(n_nodes):
    return ((n_nodes + 1 + 8 * _NS - 1) // (8 * _NS)) * (8 * _NS)


def _cnt_call(dst, et, *, n_nodes, r_cnt, k_chunk):
    e_cnt = dst.shape[0]
    ns, lanes = _NS, _LANES
    np_rows = _pad_rows(n_nodes)
    rows_per_tile = np_rows // ns
    ep = e_cnt // ns
    n_chunks = ep // k_chunk
    n_full = rows_per_tile // k_chunk
    rem = rows_per_tile - n_full * k_chunk
    assert ep % k_chunk == 0 and k_chunk % lanes == 0 and rem % 8 == 0

    mesh = plsc.VectorSubcoreMesh(core_axis_name="c", subcore_axis_name="s",
                                  num_cores=r_cnt, num_subcores=ns)

    cw = 128

    @functools.partial(
        pl.kernel,
        out_type=jax.ShapeDtypeStruct((r_cnt, np_rows, cw), jnp.float32),
        mesh=mesh,
        scratch_types=[
            pltpu.VMEM_SHARED((np_rows, cw), jnp.float32),
            pltpu.VMEM((k_chunk,), jnp.int32),
            pltpu.VMEM((k_chunk,), jnp.int32),
            pltpu.VMEM((k_chunk, cw), jnp.float32),
        ],
    )
    def cnt_kernel(dst_h, et_h, cnt_h, cacc, dv, ev, vbuf):
        c = lax.axis_index("c")
        s = lax.axis_index("s")
        row0 = s * rows_per_tile

        def fill(val):
            def body(i, carry):
                for j in range(cw // lanes):
                    vbuf[i, pl.ds(j * lanes, lanes)] = jnp.full(
                        (lanes,), val, jnp.float32)
                return carry
            lax.fori_loop(0, k_chunk, body, 0)

        fill(0.0)

        def zero_acc(i, carry):
            pltpu.sync_copy(vbuf, cacc.at[pl.ds(row0 + i * k_chunk, k_chunk)])
            return carry
        lax.fori_loop(0, n_full, zero_acc, 0)
        if rem:
            pltpu.sync_copy(vbuf.at[pl.ds(0, rem)],
                            cacc.at[pl.ds(row0 + n_full * k_chunk, rem)])

        fill(1.0)
        plsc.subcore_barrier()

        base = s * ep

        def chunk(kk, carry):
            off = base + kk * k_chunk
            pltpu.sync_copy(dst_h.at[pl.ds(off, k_chunk)], dv)
            pltpu.sync_copy(et_h.at[pl.ds(off, k_chunk)], ev)

            def idx_grp(g, carry2):
                sl = pl.ds(g * lanes, lanes)
                ev[sl] = jnp.where(ev[sl] == c, dv[sl], n_nodes)
                return carry2
            lax.fori_loop(0, k_chunk // lanes, idx_grp, 0)

            pltpu.sync_copy(vbuf, cacc.at[ev], add=True)
            return carry
        lax.fori_loop(0, n_chunks, chunk, 0)

        plsc.subcore_barrier()

        def copy_out(i, carry):
            rr = row0 + i * k_chunk
            pltpu.sync_copy(cacc.at[pl.ds(rr, k_chunk)], vbuf)
            pltpu.sync_copy(vbuf, cnt_h.at[c, pl.ds(rr, k_chunk)])
            return carry
        lax.fori_loop(0, n_full, copy_out, 0)
        if rem:
            rr = row0 + n_full * k_chunk
            pltpu.sync_copy(cacc.at[pl.ds(rr, rem)], vbuf.at[pl.ds(0, rem)])
            pltpu.sync_copy(vbuf.at[pl.ds(0, rem)],
                            cnt_h.at[c, pl.ds(rr, rem)])

    return cnt_kernel(dst, et)


def _edge_call(src, dst, et, xt_flat, f_flat, *, n_nodes, r_cnt, hdim, k_chunk):
    e_cnt = src.shape[0]
    ns, lanes = _NS, _LANES
    np_rows = _pad_rows(n_nodes)
    rows_per_tile = np_rows // ns
    ep = e_cnt // ns
    n_chunks = ep // k_chunk
    n_full = rows_per_tile // k_chunk
    rem = rows_per_tile - n_full * k_chunk
    assert ep % k_chunk == 0 and k_chunk % lanes == 0 and rem % 8 == 0

    mesh = plsc.VectorSubcoreMesh(core_axis_name="c", subcore_axis_name="s",
                                  num_cores=r_cnt, num_subcores=ns)

    @functools.partial(
        pl.kernel,
        out_type=jax.ShapeDtypeStruct((r_cnt, np_rows, hdim), jnp.float32),
        mesh=mesh,
        scratch_types=[
            pltpu.VMEM_SHARED((np_rows, hdim), jnp.float32),
            pltpu.VMEM((k_chunk,), jnp.int32),
            pltpu.VMEM((k_chunk,), jnp.int32),
            pltpu.VMEM((k_chunk,), jnp.int32),
            pltpu.VMEM((k_chunk, hdim), jnp.float32),
            pltpu.VMEM((k_chunk // 2, 2 * hdim), jnp.float32),
        ],
    )
    def edge_kernel(src_h, dst_h, et_h, xt_h, f_h, ssum_h,
                    acc, sv, dv, ev, xbuf, fbuf):
        c = lax.axis_index("c")
        s = lax.axis_index("s")
        row0 = s * rows_per_tile

        def fill_zero(i, carry):
            for j in range(hdim // lanes):
                xbuf[i, pl.ds(j * lanes, lanes)] = jnp.zeros((lanes,),
                                                             jnp.float32)
            return carry
        lax.fori_loop(0, k_chunk, fill_zero, 0)

        def zero_acc(i, carry):
            pltpu.sync_copy(xbuf, acc.at[pl.ds(row0 + i * k_chunk, k_chunk)])
            return carry
        lax.fori_loop(0, n_full, zero_acc, 0)
        if rem:
            pltpu.sync_copy(xbuf.at[pl.ds(0, rem)],
                            acc.at[pl.ds(row0 + n_full * k_chunk, rem)])

        plsc.subcore_barrier()

        base = s * ep
        c_n = c * n_nodes

        def chunk(kk, carry):
            off = base + kk * k_chunk
            pltpu.sync_copy(src_h.at[pl.ds(off, k_chunk)], sv)
            pltpu.sync_copy(dst_h.at[pl.ds(off, k_chunk)], dv)
            pltpu.sync_copy(et_h.at[pl.ds(off, k_chunk)], ev)

            def idx_grp(g, carry2):
                sl = pl.ds(g * lanes, lanes)
                s16 = sv[sl]
                d16 = dv[sl]
                e16 = ev[sl]
                sv[sl] = s16 + c_n
                dv[sl] = d16 + c_n
                ev[sl] = jnp.where(e16 == c, d16, n_nodes)
                return carry2
            lax.fori_loop(0, k_chunk // lanes, idx_grp, 0)

            pltpu.sync_copy(xt_h.at[sv], xbuf)

            half = k_chunk // 2
            for hh in range(2):
                pltpu.sync_copy(f_h.at[dv.at[pl.ds(hh * half, half)]], fbuf)

                def msg(e, carry2):
                    for j in range(hdim // lanes):
                        sl = pl.ds(j * lanes, lanes)
                        x16 = xbuf[hh * half + e, sl]
                        b16 = fbuf[e, sl]
                        g16 = fbuf[e, pl.ds(hdim + j * lanes, lanes)]
                        xbuf[hh * half + e, sl] = jnp.maximum(
                            g16 * x16 + b16, 0.0)
                    return carry2
                lax.fori_loop(0, half, msg, 0)

            pltpu.sync_copy(xbuf, acc.at[ev], add=True)
            return carry
        lax.fori_loop(0, n_chunks, chunk, 0)

        plsc.subcore_barrier()

        def copy_out(i, carry):
            rr = row0 + i * k_chunk
            pltpu.sync_copy(acc.at[pl.ds(rr, k_chunk)], xbuf)
            pltpu.sync_copy(xbuf, ssum_h.at[c, pl.ds(rr, k_chunk)])
            return carry
        lax.fori_loop(0, n_full, copy_out, 0)
        if rem:
            rr = row0 + n_full * k_chunk
            pltpu.sync_copy(acc.at[pl.ds(rr, rem)], xbuf.at[pl.ds(0, rem)])
            pltpu.sync_copy(xbuf.at[pl.ds(0, rem)],
                            ssum_h.at[c, pl.ds(rr, rem)])

    return edge_kernel(src, dst, et, xt_flat, f_flat)


def _combine_call(skip, ssum, cnt, gamma, beta, rm, rv, *, bn):
    n, hdim = skip.shape
    r_cnt, np_rows, lanes = cnt.shape

    def body(skip_ref, ss_ref, ct_ref, g_ref, b_ref, rm_ref, rv_ref, out_ref):
        out = skip_ref[...]
        for r in range(r_cnt):
            c = ct_ref[r][:, 0:1]
            out = out + ss_ref[r] * (1.0 / jnp.maximum(c, 1.0))
        scale = g_ref[...] * lax.rsqrt(rv_ref[...] + _EPS)
        out_ref[...] = (out - rm_ref[...]) * scale + b_ref[...]

    return pl.pallas_call(
        body,
        grid=(n // bn,),
        in_specs=[
            pl.BlockSpec((bn, hdim), lambda i: (i, 0)),
            pl.BlockSpec((r_cnt, bn, hdim), lambda i: (0, i, 0)),
            pl.BlockSpec((r_cnt, bn, lanes), lambda i: (0, i, 0)),
            pl.BlockSpec((1, hdim), lambda i: (0, 0)),
            pl.BlockSpec((1, hdim), lambda i: (0, 0)),
            pl.BlockSpec((1, hdim), lambda i: (0, 0)),
            pl.BlockSpec((1, hdim), lambda i: (0, 0)),
        ],
        out_specs=pl.BlockSpec((bn, hdim), lambda i: (i, 0)),
        out_shape=jax.ShapeDtypeStruct((n, hdim), jnp.float32),
    )(skip, ssum, cnt, gamma, beta, rm, rv)


def _mlp_call(h, w1, b1, w2, b2, *, bn):
    n, hdim = h.shape
    mid = w1.shape[1]

    def body(h_ref, w1_ref, b1_ref, w2_ref, b2_ref, out_ref):
        y = jnp.dot(h_ref[...], w1_ref[...],
                    preferred_element_type=jnp.float32) + b1_ref[...]
        y = jnp.where(y > 0, y, 0.2 * y)
        out_ref[...] = jnp.dot(y, w2_ref[...],
                               preferred_element_type=jnp.float32) + b2_ref[...]

    return pl.pallas_call(
        body,
        grid=(n // bn,),
        in_specs=[
            pl.BlockSpec((bn, hdim), lambda i: (i, 0)),
            pl.BlockSpec((hdim, mid), lambda i: (0, 0)),
            pl.BlockSpec((1, mid), lambda i: (0, 0)),
            pl.BlockSpec((mid, hdim), lambda i: (0, 0)),
            pl.BlockSpec((1, hdim), lambda i: (0, 0)),
        ],
        out_specs=pl.BlockSpec((bn, hdim), lambda i: (i, 0)),
        out_shape=jax.ShapeDtypeStruct((n, hdim), jnp.float32),
    )(h, w1, b1, w2, b2)


def kernel(x, edge_index, edge_type, lins_W, films_W, films_b, lin_skip_W,
           film_skip_W, bn_gamma, bn_beta, bn_rm, bn_rv, lin1_W, lin1_b,
           lin2_W, lin2_b):
    n, d = x.shape
    l_cnt, r_cnt, _, hdim = lins_W.shape
    src = edge_index[0]
    dst = edge_index[1]
    bn = 2000

    cnt = _cnt_call(dst, edge_type, n_nodes=n, r_cnt=r_cnt, k_chunk=80)
    h = x
    for l in range(l_cnt):
        skip, xt, f = _dense_call(
            h, lin_skip_W[l], film_skip_W[l], lins_W[l], films_W[l],
            films_b[l], bn=bn)
        ssum = _edge_call(
            src, dst, edge_type,
            xt.reshape(r_cnt * n, hdim), f.reshape(r_cnt * n, 2 * hdim),
            n_nodes=n, r_cnt=r_cnt, hdim=hdim, k_chunk=80)
        h = _combine_call(
            skip, ssum, cnt,
            bn_gamma[l].reshape(1, hdim), bn_beta[l].reshape(1, hdim),
            bn_rm[l].reshape(1, hdim), bn_rv[l].reshape(1, hdim), bn=bn)
    return _mlp_call(h, lin1_W, lin1_b.reshape(1, -1),
                     lin2_W, lin2_b.reshape(1, -1), bn=bn)


# speedup vs baseline: 2.8878x; 1.2576x over previous
import functools

import jax
import jax.numpy as jnp
from jax import lax
from jax.experimental import pallas as pl
from jax.experimental.pallas import tpu as pltpu
from jax.experimental.pallas import tpu_sc as plsc

_EPS = 1e-5


def _dense_call(h, lsW, fsW, linsW, filmsW, filmsb, *, bn):
    n, d = h.shape
    hdim = lsW.shape[1]
    r_cnt = linsW.shape[0]

    def body(h_ref, ls_ref, fs_ref, lin_ref, film_ref, fb_ref,
             skip_ref, xt_ref, f_ref):
        hb = h_ref[...]
        ls = jnp.dot(hb, ls_ref[...], preferred_element_type=jnp.float32)
        fs = jnp.dot(hb, fs_ref[...], preferred_element_type=jnp.float32)
        skip_ref[...] = jnp.maximum(fs[:, hdim:] * ls + fs[:, :hdim], 0.0)
        for r in range(r_cnt):
            xt_ref[r] = jnp.dot(hb, lin_ref[r],
                                preferred_element_type=jnp.float32)
            f_ref[r] = jnp.dot(hb, film_ref[r],
                               preferred_element_type=jnp.float32) + fb_ref[r]

    return pl.pallas_call(
        body,
        grid=(n // bn,),
        in_specs=[
            pl.BlockSpec((bn, d), lambda i: (i, 0)),
            pl.BlockSpec((d, hdim), lambda i: (0, 0)),
            pl.BlockSpec((d, 2 * hdim), lambda i: (0, 0)),
            pl.BlockSpec((r_cnt, d, hdim), lambda i: (0, 0, 0)),
            pl.BlockSpec((r_cnt, d, 2 * hdim), lambda i: (0, 0, 0)),
            pl.BlockSpec((r_cnt, 2 * hdim), lambda i: (0, 0)),
        ],
        out_specs=[
            pl.BlockSpec((bn, hdim), lambda i: (i, 0)),
            pl.BlockSpec((r_cnt, bn, hdim), lambda i: (0, i, 0)),
            pl.BlockSpec((r_cnt, bn, 2 * hdim), lambda i: (0, i, 0)),
        ],
        out_shape=[
            jax.ShapeDtypeStruct((n, hdim), jnp.float32),
            jax.ShapeDtypeStruct((r_cnt, n, hdim), jnp.float32),
            jax.ShapeDtypeStruct((r_cnt, n, 2 * hdim), jnp.float32),
        ],
    )(h, lsW, fsW, linsW, filmsW, filmsb)


_NS = 16
_LANES = 16


def _pad_rows(n_nodes):
    return ((n_nodes + 1 + 8 * _NS - 1) // (8 * _NS)) * (8 * _NS)


def _cnt_call(dst, et, *, n_nodes, r_cnt, k_chunk):
    e_cnt = dst.shape[0]
    ns, lanes = _NS, _LANES
    np_rows = _pad_rows(n_nodes)
    rows_per_tile = np_rows // ns
    ep = e_cnt // ns
    n_chunks = ep // k_chunk
    n_full = rows_per_tile // k_chunk
    rem = rows_per_tile - n_full * k_chunk
    assert ep % k_chunk == 0 and k_chunk % lanes == 0 and rem % 8 == 0

    mesh = plsc.VectorSubcoreMesh(core_axis_name="c", subcore_axis_name="s",
                                  num_cores=r_cnt, num_subcores=ns)

    cw = 128

    @functools.partial(
        pl.kernel,
        out_type=jax.ShapeDtypeStruct((r_cnt, np_rows, cw), jnp.float32),
        mesh=mesh,
        scratch_types=[
            pltpu.VMEM_SHARED((np_rows, cw), jnp.float32),
            pltpu.VMEM((k_chunk,), jnp.int32),
            pltpu.VMEM((k_chunk,), jnp.int32),
            pltpu.VMEM((k_chunk, cw), jnp.float32),
        ],
    )
    def cnt_kernel(dst_h, et_h, cnt_h, cacc, dv, ev, vbuf):
        c = lax.axis_index("c")
        s = lax.axis_index("s")
        row0 = s * rows_per_tile

        def fill(val):
            def body(i, carry):
                for j in range(cw // lanes):
                    vbuf[i, pl.ds(j * lanes, lanes)] = jnp.full(
                        (lanes,), val, jnp.float32)
                return carry
            lax.fori_loop(0, k_chunk, body, 0)

        fill(0.0)

        def zero_acc(i, carry):
            pltpu.sync_copy(vbuf, cacc.at[pl.ds(row0 + i * k_chunk, k_chunk)])
            return carry
        lax.fori_loop(0, n_full, zero_acc, 0)
        if rem:
            pltpu.sync_copy(vbuf.at[pl.ds(0, rem)],
                            cacc.at[pl.ds(row0 + n_full * k_chunk, rem)])

        fill(1.0)
        plsc.subcore_barrier()

        base = s * ep

        def chunk(kk, carry):
            off = base + kk * k_chunk
            pltpu.sync_copy(dst_h.at[pl.ds(off, k_chunk)], dv)
            pltpu.sync_copy(et_h.at[pl.ds(off, k_chunk)], ev)

            def idx_grp(g, carry2):
                sl = pl.ds(g * lanes, lanes)
                ev[sl] = jnp.where(ev[sl] == c, dv[sl], n_nodes)
                return carry2
            lax.fori_loop(0, k_chunk // lanes, idx_grp, 0)

            pltpu.sync_copy(vbuf, cacc.at[ev], add=True)
            return carry
        lax.fori_loop(0, n_chunks, chunk, 0)

        plsc.subcore_barrier()

        def copy_out(i, carry):
            rr = row0 + i * k_chunk
            pltpu.sync_copy(cacc.at[pl.ds(rr, k_chunk)], vbuf)
            pltpu.sync_copy(vbuf, cnt_h.at[c, pl.ds(rr, k_chunk)])
            return carry
        lax.fori_loop(0, n_full, copy_out, 0)
        if rem:
            rr = row0 + n_full * k_chunk
            pltpu.sync_copy(cacc.at[pl.ds(rr, rem)], vbuf.at[pl.ds(0, rem)])
            pltpu.sync_copy(vbuf.at[pl.ds(0, rem)],
                            cnt_h.at[c, pl.ds(rr, rem)])

    return cnt_kernel(dst, et)


def _edge_call(srca, dsta, et, xt_flat, f_flat, *, n_nodes, r_cnt, hdim,
               k_chunk):
    e_cnt = srca.shape[0]
    ns, lanes = _NS, _LANES
    np_rows = _pad_rows(n_nodes)
    rows_per_tile = np_rows // ns
    ep = e_cnt // ns
    n_chunks = ep // k_chunk
    n_full = rows_per_tile // k_chunk
    rem = rows_per_tile - n_full * k_chunk
    assert ep % k_chunk == 0 and k_chunk % lanes == 0 and rem % 8 == 0
    assert n_chunks % 2 == 0

    mesh = plsc.VectorSubcoreMesh(core_axis_name="c", subcore_axis_name="s",
                                  num_cores=r_cnt, num_subcores=ns)

    idx3 = pltpu.VMEM((3, k_chunk), jnp.int32)

    @functools.partial(
        pl.kernel,
        out_type=jax.ShapeDtypeStruct((r_cnt, np_rows, hdim), jnp.float32),
        mesh=mesh,
        scratch_types=[
            pltpu.VMEM_SHARED((np_rows, hdim), jnp.float32),
            idx3,
            idx3,
            pltpu.VMEM((k_chunk,), jnp.int32),
            pltpu.VMEM((k_chunk, hdim), jnp.float32),
            pltpu.VMEM((k_chunk // 2, 2 * hdim), jnp.float32),
            pltpu.SemaphoreType.DMA,
            pltpu.SemaphoreType.DMA,
            pltpu.SemaphoreType.DMA,
            pltpu.SemaphoreType.DMA,
        ],
    )
    def edge_kernel(src_h, dst_h, et_h, xt_h, f_h, ssum_h,
                    acc, stg_a, stg_b, ev, xbuf, fbuf,
                    sem_a, sem_b, sem_x, sem_f):
        c = lax.axis_index("c")
        s = lax.axis_index("s")
        row0 = s * rows_per_tile

        def fill_zero(i, carry):
            for j in range(hdim // lanes):
                xbuf[i, pl.ds(j * lanes, lanes)] = jnp.zeros((lanes,),
                                                             jnp.float32)
            return carry
        lax.fori_loop(0, k_chunk, fill_zero, 0)

        def zero_acc(i, carry):
            pltpu.sync_copy(xbuf, acc.at[pl.ds(row0 + i * k_chunk, k_chunk)])
            return carry
        lax.fori_loop(0, n_full, zero_acc, 0)
        if rem:
            pltpu.sync_copy(xbuf.at[pl.ds(0, rem)],
                            acc.at[pl.ds(row0 + n_full * k_chunk, rem)])

        plsc.subcore_barrier()

        base = s * ep
        c_n = c * n_nodes
        max_off = base + (n_chunks - 1) * k_chunk

        def stage_start(kk, stg, sem):
            off = jnp.minimum(base + kk * k_chunk, max_off)
            for i, arr in enumerate((src_h, dst_h, et_h)):
                pltpu.make_async_copy(arr.at[pl.ds(off, k_chunk)],
                                      stg.at[i], sem).start()

        def stage_wait(stg, sem):
            for i, arr in enumerate((src_h, dst_h, et_h)):
                pltpu.make_async_copy(arr.at[pl.ds(0, k_chunk)],
                                      stg.at[i], sem).wait()

        def process(kk, stg, sem, nstg, nsem):
            stage_wait(stg, sem)
            stage_start(kk + 1, nstg, nsem)

            sv = stg.at[0]
            dv = stg.at[1]
            tv = stg.at[2]

            def idx_grp(g, carry2):
                sl = pl.ds(g * lanes, lanes)
                m = tv[sl] == c
                d16 = dv[sl]
                sv[sl] = sv[sl] + c_n
                dv[sl] = d16 + c_n
                ev[pl.ds(g * lanes, lanes)] = jnp.where(m, d16, n_nodes)
                return carry2
            lax.fori_loop(0, k_chunk // lanes, idx_grp, 0)

            cp_x = pltpu.make_async_copy(xt_h.at[sv], xbuf, sem_x)
            cp_x.start()

            half = k_chunk // 2
            for hh in range(2):
                cp_f = pltpu.make_async_copy(
                    f_h.at[dv.at[pl.ds(hh * half, half)]], fbuf, sem_f)
                cp_f.start()
                if hh == 0:
                    cp_x.wait()
                cp_f.wait()

                def msg(e, carry2):
                    for j in range(hdim // lanes):
                        sl = pl.ds(j * lanes, lanes)
                        x16 = xbuf[hh * half + e, sl]
                        b16 = fbuf[e, sl]
                        g16 = fbuf[e, pl.ds(hdim + j * lanes, lanes)]
                        xbuf[hh * half + e, sl] = jnp.maximum(
                            g16 * x16 + b16, 0.0)
                    return carry2
                lax.fori_loop(0, half, msg, 0)

            pltpu.sync_copy(xbuf, acc.at[ev], add=True)

        stage_start(0, stg_a, sem_a)

        def chunk_pair(kk2, carry):
            process(2 * kk2, stg_a, sem_a, stg_b, sem_b)
            process(2 * kk2 + 1, stg_b, sem_b, stg_a, sem_a)
            return carry
        lax.fori_loop(0, n_chunks // 2, chunk_pair, 0)

        stage_wait(stg_a, sem_a)

        plsc.subcore_barrier()

        def copy_out(i, carry):
            rr = row0 + i * k_chunk
            pltpu.sync_copy(acc.at[pl.ds(rr, k_chunk)], xbuf)
            pltpu.sync_copy(xbuf, ssum_h.at[c, pl.ds(rr, k_chunk)])
            return carry
        lax.fori_loop(0, n_full, copy_out, 0)
        if rem:
            rr = row0 + n_full * k_chunk
            pltpu.sync_copy(acc.at[pl.ds(rr, rem)], xbuf.at[pl.ds(0, rem)])
            pltpu.sync_copy(xbuf.at[pl.ds(0, rem)],
                            ssum_h.at[c, pl.ds(rr, rem)])

    return edge_kernel(srca, dsta, et, xt_flat, f_flat)


def _combine_call(skip, ssum, cnt, gamma, beta, rm, rv, *, bn):
    n, hdim = skip.shape
    r_cnt, np_rows, lanes = cnt.shape

    def body(skip_ref, ss_ref, ct_ref, g_ref, b_ref, rm_ref, rv_ref, out_ref):
        out = skip_ref[...]
        for r in range(r_cnt):
            c = ct_ref[r][:, 0:1]
            out = out + ss_ref[r] * (1.0 / jnp.maximum(c, 1.0))
        scale = g_ref[...] * lax.rsqrt(rv_ref[...] + _EPS)
        out_ref[...] = (out - rm_ref[...]) * scale + b_ref[...]

    return pl.pallas_call(
        body,
        grid=(n // bn,),
        in_specs=[
            pl.BlockSpec((bn, hdim), lambda i: (i, 0)),
            pl.BlockSpec((r_cnt, bn, hdim), lambda i: (0, i, 0)),
            pl.BlockSpec((r_cnt, bn, lanes), lambda i: (0, i, 0)),
            pl.BlockSpec((1, hdim), lambda i: (0, 0)),
            pl.BlockSpec((1, hdim), lambda i: (0, 0)),
            pl.BlockSpec((1, hdim), lambda i: (0, 0)),
            pl.BlockSpec((1, hdim), lambda i: (0, 0)),
        ],
        out_specs=pl.BlockSpec((bn, hdim), lambda i: (i, 0)),
        out_shape=jax.ShapeDtypeStruct((n, hdim), jnp.float32),
    )(skip, ssum, cnt, gamma, beta, rm, rv)


def _mlp_call(h, w1, b1, w2, b2, *, bn):
    n, hdim = h.shape
    mid = w1.shape[1]

    def body(h_ref, w1_ref, b1_ref, w2_ref, b2_ref, out_ref):
        y = jnp.dot(h_ref[...], w1_ref[...],
                    preferred_element_type=jnp.float32) + b1_ref[...]
        y = jnp.where(y > 0, y, 0.2 * y)
        out_ref[...] = jnp.dot(y, w2_ref[...],
                               preferred_element_type=jnp.float32) + b2_ref[...]

    return pl.pallas_call(
        body,
        grid=(n // bn,),
        in_specs=[
            pl.BlockSpec((bn, hdim), lambda i: (i, 0)),
            pl.BlockSpec((hdim, mid), lambda i: (0, 0)),
            pl.BlockSpec((1, mid), lambda i: (0, 0)),
            pl.BlockSpec((mid, hdim), lambda i: (0, 0)),
            pl.BlockSpec((1, hdim), lambda i: (0, 0)),
        ],
        out_specs=pl.BlockSpec((bn, hdim), lambda i: (i, 0)),
        out_shape=jax.ShapeDtypeStruct((n, hdim), jnp.float32),
    )(h, w1, b1, w2, b2)


def kernel(x, edge_index, edge_type, lins_W, films_W, films_b, lin_skip_W,
           film_skip_W, bn_gamma, bn_beta, bn_rm, bn_rv, lin1_W, lin1_b,
           lin2_W, lin2_b):
    n, d = x.shape
    l_cnt, r_cnt, _, hdim = lins_W.shape
    src = edge_index[0]
    dst = edge_index[1]
    bn = 2000

    cnt = _cnt_call(dst, edge_type, n_nodes=n, r_cnt=r_cnt, k_chunk=80)
    h = x
    for l in range(l_cnt):
        skip, xt, f = _dense_call(
            h, lin_skip_W[l], film_skip_W[l], lins_W[l], films_W[l],
            films_b[l], bn=bn)
        ssum = _edge_call(
            src, dst, edge_type,
            xt.reshape(r_cnt * n, hdim), f.reshape(r_cnt * n, 2 * hdim),
            n_nodes=n, r_cnt=r_cnt, hdim=hdim, k_chunk=80)
        h = _combine_call(
            skip, ssum, cnt,
            bn_gamma[l].reshape(1, hdim), bn_beta[l].reshape(1, hdim),
            bn_rm[l].reshape(1, hdim), bn_rv[l].reshape(1, hdim), bn=bn)
    return _mlp_call(h, lin1_W, lin1_b.reshape(1, -1),
                     lin2_W, lin2_b.reshape(1, -1), bn=bn)


# speedup vs baseline: 3.1492x; 1.0905x over previous
import functools

import jax
import jax.numpy as jnp
from jax import lax
from jax.experimental import pallas as pl
from jax.experimental.pallas import tpu as pltpu
from jax.experimental.pallas import tpu_sc as plsc

_EPS = 1e-5


def _dense_call(h, lsW, fsW, linsW, filmsW, filmsb, *, bn):
    n, d = h.shape
    hdim = lsW.shape[1]
    r_cnt = linsW.shape[0]

    def body(h_ref, ls_ref, fs_ref, lin_ref, film_ref, fb_ref,
             skip_ref, xt_ref, f_ref):
        hb = h_ref[...]
        ls = jnp.dot(hb, ls_ref[...], preferred_element_type=jnp.float32)
        fs = jnp.dot(hb, fs_ref[...], preferred_element_type=jnp.float32)
        skip_ref[...] = jnp.maximum(fs[:, hdim:] * ls + fs[:, :hdim], 0.0)
        for r in range(r_cnt):
            xt_ref[r] = jnp.dot(hb, lin_ref[r],
                                preferred_element_type=jnp.float32)
            f_ref[r] = jnp.dot(hb, film_ref[r],
                               preferred_element_type=jnp.float32) + fb_ref[r]

    return pl.pallas_call(
        body,
        grid=(n // bn,),
        in_specs=[
            pl.BlockSpec((bn, d), lambda i: (i, 0)),
            pl.BlockSpec((d, hdim), lambda i: (0, 0)),
            pl.BlockSpec((d, 2 * hdim), lambda i: (0, 0)),
            pl.BlockSpec((r_cnt, d, hdim), lambda i: (0, 0, 0)),
            pl.BlockSpec((r_cnt, d, 2 * hdim), lambda i: (0, 0, 0)),
            pl.BlockSpec((r_cnt, 2 * hdim), lambda i: (0, 0)),
        ],
        out_specs=[
            pl.BlockSpec((bn, hdim), lambda i: (i, 0)),
            pl.BlockSpec((r_cnt, bn, hdim), lambda i: (0, i, 0)),
            pl.BlockSpec((r_cnt, bn, 2 * hdim), lambda i: (0, i, 0)),
        ],
        out_shape=[
            jax.ShapeDtypeStruct((n, hdim), jnp.float32),
            jax.ShapeDtypeStruct((r_cnt, n, hdim), jnp.float32),
            jax.ShapeDtypeStruct((r_cnt, n, 2 * hdim), jnp.float32),
        ],
    )(h, lsW, fsW, linsW, filmsW, filmsb)


_NS = 16
_LANES = 16


def _pad_rows(n_nodes):
    return ((n_nodes + 1 + 8 * _NS - 1) // (8 * _NS)) * (8 * _NS)


def _cnt_call(dst, et, *, n_nodes, r_cnt, k_chunk):
    e_cnt = dst.shape[0]
    ns, lanes = _NS, _LANES
    np_rows = _pad_rows(n_nodes)
    rows_per_tile = np_rows // ns
    ep = e_cnt // ns
    n_chunks = ep // k_chunk
    n_full = rows_per_tile // k_chunk
    rem = rows_per_tile - n_full * k_chunk
    assert ep % k_chunk == 0 and k_chunk % lanes == 0 and rem % 8 == 0

    mesh = plsc.VectorSubcoreMesh(core_axis_name="c", subcore_axis_name="s",
                                  num_cores=r_cnt, num_subcores=ns)

    cw = 128

    @functools.partial(
        pl.kernel,
        out_type=jax.ShapeDtypeStruct((r_cnt, np_rows, cw), jnp.float32),
        mesh=mesh,
        scratch_types=[
            pltpu.VMEM_SHARED((np_rows, cw), jnp.float32),
            pltpu.VMEM((k_chunk,), jnp.int32),
            pltpu.VMEM((k_chunk,), jnp.int32),
            pltpu.VMEM((k_chunk, cw), jnp.float32),
        ],
    )
    def cnt_kernel(dst_h, et_h, cnt_h, cacc, dv, ev, vbuf):
        c = lax.axis_index("c")
        s = lax.axis_index("s")
        row0 = s * rows_per_tile

        def fill(val):
            def body(i, carry):
                for j in range(cw // lanes):
                    vbuf[i, pl.ds(j * lanes, lanes)] = jnp.full(
                        (lanes,), val, jnp.float32)
                return carry
            lax.fori_loop(0, k_chunk, body, 0)

        fill(0.0)

        def zero_acc(i, carry):
            pltpu.sync_copy(vbuf, cacc.at[pl.ds(row0 + i * k_chunk, k_chunk)])
            return carry
        lax.fori_loop(0, n_full, zero_acc, 0)
        if rem:
            pltpu.sync_copy(vbuf.at[pl.ds(0, rem)],
                            cacc.at[pl.ds(row0 + n_full * k_chunk, rem)])

        fill(1.0)
        plsc.subcore_barrier()

        base = s * ep

        def chunk(kk, carry):
            off = base + kk * k_chunk
            pltpu.sync_copy(dst_h.at[pl.ds(off, k_chunk)], dv)
            pltpu.sync_copy(et_h.at[pl.ds(off, k_chunk)], ev)

            def idx_grp(g, carry2):
                sl = pl.ds(g * lanes, lanes)
                ev[sl] = jnp.where(ev[sl] == c, dv[sl], n_nodes)
                return carry2
            lax.fori_loop(0, k_chunk // lanes, idx_grp, 0)

            pltpu.sync_copy(vbuf, cacc.at[ev], add=True)
            return carry
        lax.fori_loop(0, n_chunks, chunk, 0)

        plsc.subcore_barrier()

        def copy_out(i, carry):
            rr = row0 + i * k_chunk
            pltpu.sync_copy(cacc.at[pl.ds(rr, k_chunk)], vbuf)
            pltpu.sync_copy(vbuf, cnt_h.at[c, pl.ds(rr, k_chunk)])
            return carry
        lax.fori_loop(0, n_full, copy_out, 0)
        if rem:
            rr = row0 + n_full * k_chunk
            pltpu.sync_copy(cacc.at[pl.ds(rr, rem)], vbuf.at[pl.ds(0, rem)])
            pltpu.sync_copy(vbuf.at[pl.ds(0, rem)],
                            cnt_h.at[c, pl.ds(rr, rem)])

    return cnt_kernel(dst, et)


def _edge_call(srca, dsta, et, xt_flat, f_flat, *, n_nodes, r_cnt, hdim,
               k_chunk):
    e_cnt = srca.shape[0]
    ns, lanes = _NS, _LANES
    np_rows = _pad_rows(n_nodes)
    rows_per_tile = np_rows // ns
    ep = e_cnt // ns
    n_chunks = ep // k_chunk
    n_full = rows_per_tile // k_chunk
    rem = rows_per_tile - n_full * k_chunk
    assert ep % k_chunk == 0 and k_chunk % lanes == 0 and rem % 8 == 0
    assert n_chunks % 2 == 0

    mesh = plsc.VectorSubcoreMesh(core_axis_name="c", subcore_axis_name="s",
                                  num_cores=r_cnt, num_subcores=ns)

    idx3 = pltpu.VMEM((3, k_chunk), jnp.int32)

    @functools.partial(
        pl.kernel,
        out_type=jax.ShapeDtypeStruct((r_cnt, np_rows, hdim), jnp.float32),
        mesh=mesh,
        scratch_types=[
            pltpu.VMEM_SHARED((np_rows, hdim), jnp.float32),
            idx3,
            idx3,
            pltpu.VMEM((k_chunk,), jnp.int32),
            pltpu.VMEM((k_chunk,), jnp.int32),
            pltpu.VMEM((k_chunk, hdim), jnp.float32),
            pltpu.VMEM((k_chunk // 2, 2 * hdim), jnp.float32),
            pltpu.VMEM((k_chunk // 2, 2 * hdim), jnp.float32),
            pltpu.SemaphoreType.DMA,
            pltpu.SemaphoreType.DMA,
            pltpu.SemaphoreType.DMA,
            pltpu.SemaphoreType.DMA,
            pltpu.SemaphoreType.DMA,
            pltpu.SemaphoreType.DMA,
        ],
    )
    def edge_kernel(src_h, dst_h, et_h, xt_h, f_h, ssum_h,
                    acc, stg_a, stg_b, ev_a, ev_b, xbuf, fbuf_a, fbuf_b,
                    sem_a, sem_b, sem_x, sem_f, sem_g, sem_sc):
        c = lax.axis_index("c")
        s = lax.axis_index("s")
        row0 = s * rows_per_tile

        def fill_zero(i, carry):
            for j in range(hdim // lanes):
                xbuf[i, pl.ds(j * lanes, lanes)] = jnp.zeros((lanes,),
                                                             jnp.float32)
            return carry
        lax.fori_loop(0, k_chunk, fill_zero, 0)

        def fill_trash(g, carry):
            ev_a[pl.ds(g * lanes, lanes)] = jnp.full((lanes,), n_nodes,
                                                     jnp.int32)
            ev_b[pl.ds(g * lanes, lanes)] = jnp.full((lanes,), n_nodes,
                                                     jnp.int32)
            return carry
        lax.fori_loop(0, k_chunk // lanes, fill_trash, 0)

        def zero_acc(i, carry):
            pltpu.sync_copy(xbuf, acc.at[pl.ds(row0 + i * k_chunk, k_chunk)])
            return carry
        lax.fori_loop(0, n_full, zero_acc, 0)
        if rem:
            pltpu.sync_copy(xbuf.at[pl.ds(0, rem)],
                            acc.at[pl.ds(row0 + n_full * k_chunk, rem)])

        plsc.subcore_barrier()

        base = s * ep
        c_n = c * n_nodes
        max_off = base + (n_chunks - 1) * k_chunk

        def stage_start(kk, stg, sem):
            off = jnp.minimum(base + kk * k_chunk, max_off)
            for i, arr in enumerate((src_h, dst_h, et_h)):
                pltpu.make_async_copy(arr.at[pl.ds(off, k_chunk)],
                                      stg.at[i], sem).start()

        def stage_wait(stg, sem):
            for i, arr in enumerate((src_h, dst_h, et_h)):
                pltpu.make_async_copy(arr.at[pl.ds(0, k_chunk)],
                                      stg.at[i], sem).wait()

        def process(kk, stg, sem, nstg, nsem, ev):
            stage_wait(stg, sem)
            stage_start(kk + 1, nstg, nsem)

            sv = stg.at[0]
            dv = stg.at[1]
            tv = stg.at[2]

            def idx_grp(g, carry2):
                sl = pl.ds(g * lanes, lanes)
                m = tv[sl] == c
                d16 = dv[sl]
                sv[sl] = sv[sl] + c_n
                dv[sl] = d16 + c_n
                ev[pl.ds(g * lanes, lanes)] = jnp.where(m, d16, n_nodes)
                return carry2
            lax.fori_loop(0, k_chunk // lanes, idx_grp, 0)

            half = k_chunk // 2
            cp_x = pltpu.make_async_copy(xt_h.at[sv], xbuf, sem_x)
            cp_f0 = pltpu.make_async_copy(
                f_h.at[dv.at[pl.ds(0, half)]], fbuf_a, sem_f)
            cp_f1 = pltpu.make_async_copy(
                f_h.at[dv.at[pl.ds(half, half)]], fbuf_b, sem_g)

            pltpu.make_async_copy(xbuf, acc.at[ev_a], sem_sc).wait()
            cp_x.start()
            cp_f0.start()
            cp_f1.start()
            cp_x.wait()

            for hh, fbuf in ((0, fbuf_a), (1, fbuf_b)):
                (cp_f0 if hh == 0 else cp_f1).wait()

                def msg(e, carry2):
                    for j in range(hdim // lanes):
                        sl = pl.ds(j * lanes, lanes)
                        x16 = xbuf[hh * half + e, sl]
                        b16 = fbuf[e, sl]
                        g16 = fbuf[e, pl.ds(hdim + j * lanes, lanes)]
                        xbuf[hh * half + e, sl] = jnp.maximum(
                            g16 * x16 + b16, 0.0)
                    return carry2
                lax.fori_loop(0, half, msg, 0, unroll=2)

            pltpu.make_async_copy(xbuf, acc.at[ev], sem_sc).start(add=True)

        stage_start(0, stg_a, sem_a)
        pltpu.make_async_copy(xbuf, acc.at[ev_a], sem_sc).start(add=True)

        def chunk_pair(kk2, carry):
            process(2 * kk2, stg_a, sem_a, stg_b, sem_b, ev_a)
            process(2 * kk2 + 1, stg_b, sem_b, stg_a, sem_a, ev_b)
            return carry
        lax.fori_loop(0, n_chunks // 2, chunk_pair, 0)

        pltpu.make_async_copy(xbuf, acc.at[ev_a], sem_sc).wait()
        stage_wait(stg_a, sem_a)

        plsc.subcore_barrier()

        def copy_out(i, carry):
            rr = row0 + i * k_chunk
            pltpu.sync_copy(acc.at[pl.ds(rr, k_chunk)], xbuf)
            pltpu.sync_copy(xbuf, ssum_h.at[c, pl.ds(rr, k_chunk)])
            return carry
        lax.fori_loop(0, n_full, copy_out, 0)
        if rem:
            rr = row0 + n_full * k_chunk
            pltpu.sync_copy(acc.at[pl.ds(rr, rem)], xbuf.at[pl.ds(0, rem)])
            pltpu.sync_copy(xbuf.at[pl.ds(0, rem)],
                            ssum_h.at[c, pl.ds(rr, rem)])

    return edge_kernel(srca, dsta, et, xt_flat, f_flat)


def _combine_call(skip, ssum, cnt, gamma, beta, rm, rv, *, bn):
    n, hdim = skip.shape
    r_cnt, np_rows, lanes = cnt.shape

    def body(skip_ref, ss_ref, ct_ref, g_ref, b_ref, rm_ref, rv_ref, out_ref):
        out = skip_ref[...]
        for r in range(r_cnt):
            c = ct_ref[r][:, 0:1]
            out = out + ss_ref[r] * (1.0 / jnp.maximum(c, 1.0))
        scale = g_ref[...] * lax.rsqrt(rv_ref[...] + _EPS)
        out_ref[...] = (out - rm_ref[...]) * scale + b_ref[...]

    return pl.pallas_call(
        body,
        grid=(n // bn,),
        in_specs=[
            pl.BlockSpec((bn, hdim), lambda i: (i, 0)),
            pl.BlockSpec((r_cnt, bn, hdim), lambda i: (0, i, 0)),
            pl.BlockSpec((r_cnt, bn, lanes), lambda i: (0, i, 0)),
            pl.BlockSpec((1, hdim), lambda i: (0, 0)),
            pl.BlockSpec((1, hdim), lambda i: (0, 0)),
            pl.BlockSpec((1, hdim), lambda i: (0, 0)),
            pl.BlockSpec((1, hdim), lambda i: (0, 0)),
        ],
        out_specs=pl.BlockSpec((bn, hdim), lambda i: (i, 0)),
        out_shape=jax.ShapeDtypeStruct((n, hdim), jnp.float32),
    )(skip, ssum, cnt, gamma, beta, rm, rv)


def _mlp_call(h, w1, b1, w2, b2, *, bn):
    n, hdim = h.shape
    mid = w1.shape[1]

    def body(h_ref, w1_ref, b1_ref, w2_ref, b2_ref, out_ref):
        y = jnp.dot(h_ref[...], w1_ref[...],
                    preferred_element_type=jnp.float32) + b1_ref[...]
        y = jnp.where(y > 0, y, 0.2 * y)
        out_ref[...] = jnp.dot(y, w2_ref[...],
                               preferred_element_type=jnp.float32) + b2_ref[...]

    return pl.pallas_call(
        body,
        grid=(n // bn,),
        in_specs=[
            pl.BlockSpec((bn, hdim), lambda i: (i, 0)),
            pl.BlockSpec((hdim, mid), lambda i: (0, 0)),
            pl.BlockSpec((1, mid), lambda i: (0, 0)),
            pl.BlockSpec((mid, hdim), lambda i: (0, 0)),
            pl.BlockSpec((1, hdim), lambda i: (0, 0)),
        ],
        out_specs=pl.BlockSpec((bn, hdim), lambda i: (i, 0)),
        out_shape=jax.ShapeDtypeStruct((n, hdim), jnp.float32),
    )(h, w1, b1, w2, b2)


def kernel(x, edge_index, edge_type, lins_W, films_W, films_b, lin_skip_W,
           film_skip_W, bn_gamma, bn_beta, bn_rm, bn_rv, lin1_W, lin1_b,
           lin2_W, lin2_b):
    n, d = x.shape
    l_cnt, r_cnt, _, hdim = lins_W.shape
    src = edge_index[0]
    dst = edge_index[1]
    bn = 2000

    cnt = _cnt_call(dst, edge_type, n_nodes=n, r_cnt=r_cnt, k_chunk=80)
    h = x
    for l in range(l_cnt):
        skip, xt, f = _dense_call(
            h, lin_skip_W[l], film_skip_W[l], lins_W[l], films_W[l],
            films_b[l], bn=bn)
        ssum = _edge_call(
            src, dst, edge_type,
            xt.reshape(r_cnt * n, hdim), f.reshape(r_cnt * n, 2 * hdim),
            n_nodes=n, r_cnt=r_cnt, hdim=hdim, k_chunk=80)
        h = _combine_call(
            skip, ssum, cnt,
            bn_gamma[l].reshape(1, hdim), bn_beta[l].reshape(1, hdim),
            bn_rm[l].reshape(1, hdim), bn_rv[l].reshape(1, hdim), bn=bn)
    return _mlp_call(h, lin1_W, lin1_b.reshape(1, -1),
                     lin2_W, lin2_b.reshape(1, -1), bn=bn)
